# baseline (speedup 1.0000x reference)
import functools

import jax
import jax.numpy as jnp
from jax import lax
from jax.experimental import pallas as pl
from jax.experimental.pallas import tpu as pltpu
from jax.experimental.pallas import tpu_sc as plsc

NUM_TILES = 32
CHUNK = 128


def _sc_body(B, K, D, W, pairs_hbm, obs_hbm, samp_hbm, pair_tab, rel_tab,
             predrep_out, pos_out, neg_out,
             idx_pair, idx_obs, idx_samp, P, A, S, pos_s, neg_s):
  nc = 2
  wid = lax.axis_index("s") * nc + lax.axis_index("c")
  base = wid * W
  rows16 = lax.iota(jnp.int32, 16)
  n_chunks = W // CHUNK
  n_groups = CHUNK // 16

  for ch in range(n_chunks):
    rbase = base + ch * CHUNK
    pltpu.sync_copy(pairs_hbm.at[pl.ds(rbase, CHUNK)], idx_pair)
    pltpu.sync_copy(obs_hbm.at[pl.ds(rbase, CHUNK)], idx_obs)
    for k in range(K):
      pltpu.sync_copy(samp_hbm.at[pl.ds(k * B + rbase, CHUNK)],
                      idx_samp.at[k])
    pltpu.sync_copy(pair_tab.at[idx_pair], P)
    pltpu.sync_copy(rel_tab.at[idx_obs], A)
    for k in range(K):
      pltpu.sync_copy(rel_tab.at[idx_samp.at[k]], S.at[k])
    for k in range(K):
      pltpu.sync_copy(P, predrep_out.at[pl.ds(k * B + rbase, CHUNK)])

    def g_body(g, carry):
      row_idx = rows16 + g * 16

      def d_body(d, accs):
        col = jnp.zeros((16,), jnp.int32) + d
        pcol = plsc.load_gather(P, [row_idx, col])
        ocol = plsc.load_gather(A, [row_idx, col])
        acc_p = accs[0] + pcol * ocol
        acc_n = [accs[1 + k] + pcol * plsc.load_gather(S.at[k], [row_idx, col])
                 for k in range(K)]
        return tuple([acc_p] + acc_n)

      zero = jnp.zeros((16,), jnp.float32)
      accs = lax.fori_loop(0, D, d_body, tuple(zero for _ in range(K + 1)))
      off = ch * CHUNK + g * 16
      pos_s[pl.ds(off, 16)] = accs[0]
      for k in range(K):
        neg_s[k, pl.ds(off, 16)] = accs[1 + k]
      return carry

    lax.fori_loop(0, n_groups, g_body, 0)

  pltpu.sync_copy(pos_s, pos_out.at[pl.ds(base, W)])
  for k in range(K):
    pltpu.sync_copy(neg_s.at[k], neg_out.at[pl.ds(k * B + base, W)])


def _sc_call(pairs, obs, samp, pair_table, rel_table):
  B = pairs.shape[0]
  K = samp.shape[0] // B
  D = pair_table.shape[1]
  W = B // NUM_TILES
  mesh = plsc.VectorSubcoreMesh(core_axis_name="c", subcore_axis_name="s")
  body = functools.partial(_sc_body, B, K, D, W)
  f = pl.kernel(
      body,
      out_type=[
          jax.ShapeDtypeStruct((K * B, D), jnp.float32),
          jax.ShapeDtypeStruct((B,), jnp.float32),
          jax.ShapeDtypeStruct((K * B,), jnp.float32),
      ],
      mesh=mesh,
      compiler_params=pltpu.CompilerParams(
          needs_layout_passes=False,
          use_tc_tiling_on_sc=False,
      ),
      scratch_types=[
          pltpu.VMEM((CHUNK,), jnp.int32),
          pltpu.VMEM((CHUNK,), jnp.int32),
          pltpu.VMEM((K, CHUNK), jnp.int32),
          pltpu.VMEM((CHUNK, D), jnp.float32),
          pltpu.VMEM((CHUNK, D), jnp.float32),
          pltpu.VMEM((K, CHUNK, D), jnp.float32),
          pltpu.VMEM((W,), jnp.float32),
          pltpu.VMEM((K, W), jnp.float32),
      ],
  )
  return f(pairs, obs, samp, pair_table, rel_table)


def _tc_body(pos_ref, neg_ref, obsp_ref, sampp_ref, loss_ref, pl_ref, nl_ref):
  pos = pos_ref[...]
  neg = neg_ref[...]
  obsp_ref[...] = jax.nn.sigmoid(pos)
  sampp_ref[...] = jax.nn.sigmoid(neg)
  pos_ls = jnp.minimum(pos, 0.0) - jnp.log1p(jnp.exp(-jnp.abs(pos)))
  neg_ls = jnp.minimum(-neg, 0.0) - jnp.log1p(jnp.exp(-jnp.abs(neg)))
  p_loss = -jnp.sum(pos_ls)
  n_loss = -jnp.sum(neg_ls)
  pl_ref[0, 0] = p_loss
  nl_ref[0, 0] = n_loss
  loss_ref[0, 0] = p_loss + n_loss


def _tc_call(pos_scores, neg_scores):
  B = pos_scores.shape[0]
  KB = neg_scores.shape[0]
  pos2 = pos_scores.reshape(B // 128, 128)
  neg2 = neg_scores.reshape(KB // 128, 128)
  smem = pl.BlockSpec(memory_space=pltpu.SMEM)
  obsp, sampp, loss, pl_, nl = pl.pallas_call(
      _tc_body,
      out_shape=[
          jax.ShapeDtypeStruct(pos2.shape, jnp.float32),
          jax.ShapeDtypeStruct(neg2.shape, jnp.float32),
          jax.ShapeDtypeStruct((1, 1), jnp.float32),
          jax.ShapeDtypeStruct((1, 1), jnp.float32),
          jax.ShapeDtypeStruct((1, 1), jnp.float32),
      ],
      out_specs=[
          pl.BlockSpec(memory_space=pltpu.VMEM),
          pl.BlockSpec(memory_space=pltpu.VMEM),
          smem, smem, smem,
      ],
  )(pos2, neg2)
  return (obsp.reshape(B), sampp.reshape(KB),
          loss[0, 0], pl_[0, 0], nl[0, 0])


def kernel(pairs, observed_relations, sampled_relations, pair_table, rel_table):
  pairs = pairs.astype(jnp.int32)
  obs = observed_relations.reshape(-1).astype(jnp.int32)
  samp = sampled_relations.reshape(-1).astype(jnp.int32)
  pred_rep, pos_scores, neg_scores = _sc_call(
      pairs, obs, samp, pair_table, rel_table)
  obs_p, samp_p, loss, p_loss, n_loss = _tc_call(pos_scores, neg_scores)
  return (pred_rep, loss, p_loss, n_loss, obs_p, samp_p)


# speedup vs baseline: 1.5200x; 1.5200x over previous
import functools

import jax
import jax.numpy as jnp
from jax import lax
from jax.experimental import pallas as pl
from jax.experimental.pallas import tpu as pltpu
from jax.experimental.pallas import tpu_sc as plsc

NUM_TILES = 32
CHUNK = 128


def _sc_body(B, K, D, W, pairs_hbm, obs_hbm, samp_hbm, pair_tab, rel_tab,
             predrep_out, pos_out, neg_out,
             idx_p, idx_o, idx_s, P, A, S, pos_s, neg_s, sem):
  nc = 2
  wid = lax.axis_index("s") * nc + lax.axis_index("c")
  base = wid * W
  rows16 = lax.iota(jnp.int32, 16)
  n_chunks = W // CHUNK
  n_groups = CHUNK // 16

  for ch in range(n_chunks):
    rbase = base + ch * CHUNK
    pltpu.sync_copy(pairs_hbm.at[pl.ds(rbase, CHUNK)], idx_p)
    pltpu.sync_copy(obs_hbm.at[pl.ds(rbase, CHUNK)], idx_o)
    for k in range(K):
      pltpu.sync_copy(samp_hbm.at[pl.ds(k * B + rbase, CHUNK)],
                      idx_s.at[k])

    def g_issue(g, carry):
      vp = idx_p[pl.ds(g * 16, 16)]
      vo = idx_o[pl.ds(g * 16, 16)]
      vs = [idx_s[k, pl.ds(g * 16, 16)] for k in range(K)]
      for j in range(16):
        r = g * 16 + j
        pltpu.async_copy(pair_tab.at[pl.ds(vp[j], 1)], P.at[pl.ds(r, 1)], sem)
        pltpu.async_copy(rel_tab.at[pl.ds(vo[j], 1)], A.at[pl.ds(r, 1)], sem)
        for k in range(K):
          pltpu.async_copy(rel_tab.at[pl.ds(vs[k][j], 1)],
                           S.at[pl.ds(k * CHUNK + r, 1)], sem)
      return carry

    lax.fori_loop(0, n_groups, g_issue, 0)
    pltpu.make_async_copy(pair_tab.at[pl.ds(0, CHUNK)], P, sem).wait()
    pltpu.make_async_copy(rel_tab.at[pl.ds(0, CHUNK)], A, sem).wait()
    pltpu.make_async_copy(rel_tab.at[pl.ds(0, K * CHUNK)], S, sem).wait()

    for k in range(K):
      pltpu.sync_copy(P, predrep_out.at[pl.ds(k * B + rbase, CHUNK)])

    def g_body(g, carry):
      row_p = rows16 + g * 16

      def d_body(dd, accs):
        acc_p = accs[0]
        acc_n = list(accs[1:])
        for u in range(4):
          d = dd * 4 + u
          col = jnp.zeros((16,), jnp.int32) + d
          pcol = plsc.load_gather(P, [row_p, col])
          ocol = plsc.load_gather(A, [row_p, col])
          acc_p = acc_p + pcol * ocol
          for k in range(K):
            scol = plsc.load_gather(S, [row_p + k * CHUNK, col])
            acc_n[k] = acc_n[k] + pcol * scol
        return tuple([acc_p] + acc_n)

      zero = jnp.zeros((16,), jnp.float32)
      accs = lax.fori_loop(0, D // 4, d_body,
                           tuple(zero for _ in range(K + 1)))
      off = ch * CHUNK + g * 16
      pos_s[pl.ds(off, 16)] = accs[0]
      for k in range(K):
        neg_s[k, pl.ds(off, 16)] = accs[1 + k]
      return carry

    lax.fori_loop(0, n_groups, g_body, 0)

  pltpu.sync_copy(pos_s, pos_out.at[pl.ds(base, W)])
  for k in range(K):
    pltpu.sync_copy(neg_s.at[k], neg_out.at[pl.ds(k * B + base, W)])


def _sc_call(pairs, obs, samp, pair_table, rel_table):
  B = pairs.shape[0]
  K = samp.shape[0] // B
  D = pair_table.shape[1]
  W = B // NUM_TILES
  mesh = plsc.VectorSubcoreMesh(core_axis_name="c", subcore_axis_name="s")
  body = functools.partial(_sc_body, B, K, D, W)
  f = pl.kernel(
      body,
      out_type=[
          jax.ShapeDtypeStruct((K * B, D), jnp.float32),
          jax.ShapeDtypeStruct((B,), jnp.float32),
          jax.ShapeDtypeStruct((K * B,), jnp.float32),
      ],
      mesh=mesh,
      compiler_params=pltpu.CompilerParams(
          needs_layout_passes=False,
          use_tc_tiling_on_sc=True,
      ),
      scratch_types=[
          pltpu.VMEM((CHUNK,), jnp.int32),
          pltpu.VMEM((CHUNK,), jnp.int32),
          pltpu.VMEM((K, CHUNK), jnp.int32),
          pltpu.VMEM((CHUNK, D), jnp.float32),
          pltpu.VMEM((CHUNK, D), jnp.float32),
          pltpu.VMEM((K * CHUNK, D), jnp.float32),
          pltpu.VMEM((W,), jnp.float32),
          pltpu.VMEM((K, W), jnp.float32),
          pltpu.SemaphoreType.DMA,
      ],
  )
  return f(pairs, obs, samp, pair_table, rel_table)


def _tc_body(pos_ref, neg_ref, obsp_ref, sampp_ref, loss_ref, pl_ref, nl_ref):
  pos = pos_ref[...]
  neg = neg_ref[...]
  obsp_ref[...] = jax.nn.sigmoid(pos)
  sampp_ref[...] = jax.nn.sigmoid(neg)
  pos_ls = jnp.minimum(pos, 0.0) - jnp.log1p(jnp.exp(-jnp.abs(pos)))
  neg_ls = jnp.minimum(-neg, 0.0) - jnp.log1p(jnp.exp(-jnp.abs(neg)))
  p_loss = -jnp.sum(pos_ls)
  n_loss = -jnp.sum(neg_ls)
  pl_ref[0, 0] = p_loss
  nl_ref[0, 0] = n_loss
  loss_ref[0, 0] = p_loss + n_loss


def _tc_call(pos_scores, neg_scores):
  B = pos_scores.shape[0]
  KB = neg_scores.shape[0]
  smem = pl.BlockSpec(memory_space=pltpu.SMEM)
  obsp, sampp, loss, pl_, nl = pl.pallas_call(
      _tc_body,
      out_shape=[
          jax.ShapeDtypeStruct((B,), jnp.float32),
          jax.ShapeDtypeStruct((KB,), jnp.float32),
          jax.ShapeDtypeStruct((1, 1), jnp.float32),
          jax.ShapeDtypeStruct((1, 1), jnp.float32),
          jax.ShapeDtypeStruct((1, 1), jnp.float32),
      ],
      out_specs=[
          pl.BlockSpec(memory_space=pltpu.VMEM),
          pl.BlockSpec(memory_space=pltpu.VMEM),
          smem, smem, smem,
      ],
  )(pos_scores, neg_scores)
  return obsp, sampp, loss[0, 0], pl_[0, 0], nl[0, 0]


def kernel(pairs, observed_relations, sampled_relations, pair_table, rel_table):
  pairs = pairs.astype(jnp.int32)
  obs = observed_relations.reshape(-1).astype(jnp.int32)
  samp = sampled_relations.reshape(-1).astype(jnp.int32)
  pred_rep, pos_scores, neg_scores = _sc_call(
      pairs, obs, samp, pair_table, rel_table)
  obs_p, samp_p, loss, p_loss, n_loss = _tc_call(pos_scores, neg_scores)
  return (pred_rep, loss, p_loss, n_loss, obs_p, samp_p)


# speedup vs baseline: 1.5580x; 1.0250x over previous
import functools

import jax
import jax.numpy as jnp
from jax import lax
from jax.experimental import pallas as pl
from jax.experimental.pallas import tpu as pltpu
from jax.experimental.pallas import tpu_sc as plsc

NUM_TILES = 32
CHUNK = 128


def _sc_body(B, K, D, W, pairs_hbm, obs_hbm, samp_hbm, pair_tab, rel_tab,
             predT_out, pos_out, neg_out,
             idx_p, idx_o, idx_s, P0, P1, PT, A, S, pos_s, neg_s,
             semp, semr):
  nc = 2
  wid = lax.axis_index("s") * nc + lax.axis_index("c")
  base = wid * W
  rows16 = lax.iota(jnp.int32, 16)
  n_chunks = W // CHUNK
  n_groups = CHUNK // 16
  pbufs = [P0, P1]

  def stage_idx(ch):
    rbase = base + ch * CHUNK
    pltpu.sync_copy(pairs_hbm.at[pl.ds(rbase, CHUNK)], idx_p.at[ch])
    pltpu.sync_copy(obs_hbm.at[pl.ds(rbase, CHUNK)], idx_o.at[ch])
    for k in range(K):
      pltpu.sync_copy(samp_hbm.at[pl.ds(k * B + rbase, CHUNK)],
                      idx_s.at[ch, k])

  def issue_pair(ch):
    P = pbufs[ch % 2]

    def g_issue(g, carry):
      v = idx_p[ch, pl.ds(g * 16, 16)]
      for j in range(16):
        pltpu.async_copy(pair_tab.at[pl.ds(v[j], 1)],
                         P.at[pl.ds(g * 16 + j, 1)], semp)
      return carry

    lax.fori_loop(0, n_groups, g_issue, 0)

  def drain_pair(ch):
    pltpu.make_async_copy(pair_tab.at[pl.ds(0, CHUNK)], pbufs[ch % 2],
                          semp).wait()

  def issue_rel(ch):
    pltpu.async_copy(rel_tab.at[idx_o.at[ch]], A, semr)
    for k in range(K):
      pltpu.async_copy(rel_tab.at[idx_s.at[ch, k]],
                       S.at[pl.ds(k * CHUNK, CHUNK)], semr)

  def drain_rel():
    pltpu.make_async_copy(rel_tab.at[pl.ds(0, CHUNK)], A, semr).wait()
    pltpu.make_async_copy(rel_tab.at[pl.ds(0, K * CHUNK)], S, semr).wait()

  def compute(ch):
    P = pbufs[ch % 2]

    def g_body(g, carry):
      row_r = rows16 + g * 16

      def d_body(dd, accs):
        acc_p = accs[0]
        acc_n = list(accs[1:])
        for u in range(4):
          d = dd * 4 + u
          col = jnp.zeros((16,), jnp.int32) + d
          pcol = plsc.load_gather(P, [row_r, col])
          plsc.store_scatter(PT, [col, row_r], pcol)
          ocol = plsc.load_gather(A, [row_r, col])
          acc_p = acc_p + pcol * ocol
          for k in range(K):
            scol = plsc.load_gather(S, [row_r + k * CHUNK, col])
            acc_n[k] = acc_n[k] + pcol * scol
        return tuple([acc_p] + acc_n)

      zero = jnp.zeros((16,), jnp.float32)
      accs = lax.fori_loop(0, D // 4, d_body,
                           tuple(zero for _ in range(K + 1)))
      off = ch * CHUNK + g * 16
      pos_s[pl.ds(off, 16)] = accs[0]
      for k in range(K):
        neg_s[k, pl.ds(off, 16)] = accs[1 + k]
      return carry

    lax.fori_loop(0, n_groups, g_body, 0)
    rbase = base + ch * CHUNK
    for k in range(K):
      pltpu.sync_copy(PT, predT_out.at[:, pl.ds(k * B + rbase, CHUNK)])

  for ch in range(n_chunks):
    stage_idx(ch)
  issue_pair(0)
  for ch in range(n_chunks):
    issue_rel(ch)
    drain_pair(ch)
    if ch + 1 < n_chunks:
      issue_pair(ch + 1)
    drain_rel()
    compute(ch)

  pltpu.sync_copy(pos_s, pos_out.at[pl.ds(base, W)])
  for k in range(K):
    pltpu.sync_copy(neg_s.at[k], neg_out.at[pl.ds(k * B + base, W)])


def _sc_call(pairs, obs, samp, pair_table, rel_pad):
  B = pairs.shape[0]
  K = samp.shape[0] // B
  D = pair_table.shape[1]
  W = B // NUM_TILES
  n_chunks = W // CHUNK
  mesh = plsc.VectorSubcoreMesh(core_axis_name="c", subcore_axis_name="s")
  body = functools.partial(_sc_body, B, K, D, W)
  f = pl.kernel(
      body,
      out_type=[
          jax.ShapeDtypeStruct((D, K * B), jnp.float32),
          jax.ShapeDtypeStruct((B,), jnp.float32),
          jax.ShapeDtypeStruct((K * B,), jnp.float32),
      ],
      mesh=mesh,
      compiler_params=pltpu.CompilerParams(
          needs_layout_passes=False,
          use_tc_tiling_on_sc=True,
      ),
      scratch_types=[
          pltpu.VMEM((n_chunks, CHUNK), jnp.int32),
          pltpu.VMEM((n_chunks, CHUNK), jnp.int32),
          pltpu.VMEM((n_chunks, K, CHUNK), jnp.int32),
          pltpu.VMEM((CHUNK, D), jnp.float32),
          pltpu.VMEM((CHUNK, D), jnp.float32),
          pltpu.VMEM((D, CHUNK), jnp.float32),
          pltpu.VMEM((CHUNK, 2 * D), jnp.float32),
          pltpu.VMEM((K * CHUNK, 2 * D), jnp.float32),
          pltpu.VMEM((W,), jnp.float32),
          pltpu.VMEM((K, W), jnp.float32),
          pltpu.SemaphoreType.DMA,
          pltpu.SemaphoreType.DMA,
      ],
  )
  predT, pos, neg = f(pairs, obs, samp, pair_table, rel_pad)
  return predT.T, pos, neg


def _tc_body(pos_ref, neg_ref, obsp_ref, sampp_ref, loss_ref, pl_ref, nl_ref):
  pos = pos_ref[...]
  neg = neg_ref[...]
  obsp_ref[...] = jax.nn.sigmoid(pos)
  sampp_ref[...] = jax.nn.sigmoid(neg)
  pos_ls = jnp.minimum(pos, 0.0) - jnp.log1p(jnp.exp(-jnp.abs(pos)))
  neg_ls = jnp.minimum(-neg, 0.0) - jnp.log1p(jnp.exp(-jnp.abs(neg)))
  p_loss = -jnp.sum(pos_ls)
  n_loss = -jnp.sum(neg_ls)
  pl_ref[0, 0] = p_loss
  nl_ref[0, 0] = n_loss
  loss_ref[0, 0] = p_loss + n_loss


def _tc_call(pos_scores, neg_scores):
  B = pos_scores.shape[0]
  KB = neg_scores.shape[0]
  smem = pl.BlockSpec(memory_space=pltpu.SMEM)
  obsp, sampp, loss, pl_, nl = pl.pallas_call(
      _tc_body,
      out_shape=[
          jax.ShapeDtypeStruct((B,), jnp.float32),
          jax.ShapeDtypeStruct((KB,), jnp.float32),
          jax.ShapeDtypeStruct((1, 1), jnp.float32),
          jax.ShapeDtypeStruct((1, 1), jnp.float32),
          jax.ShapeDtypeStruct((1, 1), jnp.float32),
      ],
      out_specs=[
          pl.BlockSpec(memory_space=pltpu.VMEM),
          pl.BlockSpec(memory_space=pltpu.VMEM),
          smem, smem, smem,
      ],
  )(pos_scores, neg_scores)
  return obsp, sampp, loss[0, 0], pl_[0, 0], nl[0, 0]


def kernel(pairs, observed_relations, sampled_relations, pair_table, rel_table):
  pairs = pairs.astype(jnp.int32)
  obs = observed_relations.reshape(-1).astype(jnp.int32)
  samp = sampled_relations.reshape(-1).astype(jnp.int32)
  rel_pad = jnp.pad(rel_table, ((0, 0), (0, 128 - rel_table.shape[1])))
  pred_rep, pos_scores, neg_scores = _sc_call(
      pairs, obs, samp, pair_table, rel_pad)
  obs_p, samp_p, loss, p_loss, n_loss = _tc_call(pos_scores, neg_scores)
  return (pred_rep, loss, p_loss, n_loss, obs_p, samp_p)
